# Initial kernel scaffold; baseline (speedup 1.0000x reference)
import jax
import jax.numpy as jnp
from jax.experimental import pallas as pl


def kernel(x, edge_index, edge_vals, W_0_0, b_0_0, W_0_1, b_0_1, W_1_0, b_1_0, W_1_1, b_1_1):
    raise NotImplementedError("write your pallas kernel here")


# speedup vs baseline: 1.8449x; 1.8449x over previous
import functools

import jax
import jax.numpy as jnp
from jax import lax
from jax.experimental import pallas as pl
from jax.experimental.pallas import tpu as pltpu
from jax.experimental.pallas import tpu_sc as plsc

NC = 2
NS = 16
K = 128
G = 8

_SELU_ALPHA = 1.6732632423543772848170429916717
_SELU_SCALE = 1.0507009873554804934193349852946


def _sc_agg_body(x_hbm, rows_hbm, cols_hbm, vals_hbm, out_hbm,
                 agg_sh, rows_b, cols_b, vals_b, xrows_v, scaled_v, sem):
    n = x_hbm.shape[0]
    d = x_hbm.shape[1]
    nchunks = rows_hbm.shape[1]
    ngroups = nchunks // G
    c = lax.axis_index("c")
    s = lax.axis_index("s")
    rows_per_sub = (n // NS) // 8 * 8
    tail = n - rows_per_sub * NS
    nvec = d // 16

    def zero_body(i, _):
        for f in range(nvec):
            scaled_v[i, pl.ds(f * 16, 16)] = jnp.zeros((16,), jnp.float32)
        return 0
    lax.fori_loop(0, K, zero_body, 0)
    nfull = rows_per_sub // K
    for j in range(nfull):
        pltpu.sync_copy(scaled_v, agg_sh.at[pl.ds(s * rows_per_sub + j * K, K), :])
    rem = rows_per_sub - nfull * K
    if rem:
        pltpu.sync_copy(scaled_v.at[pl.ds(0, rem), :],
                        agg_sh.at[pl.ds(s * rows_per_sub + nfull * K, rem), :])
    @pl.when(s == NS - 1)
    def _():
        pltpu.sync_copy(scaled_v.at[pl.ds(0, tail), :],
                        agg_sh.at[pl.ds(rows_per_sub * NS, tail), :])
    plsc.subcore_barrier()

    sq = lax.broadcast(c == 1, (16,))

    def group_body(g, _):
        pltpu.sync_copy(rows_hbm.at[s, pl.ds(g * G, G), :], rows_b)
        pltpu.sync_copy(cols_hbm.at[s, pl.ds(g * G, G), :], cols_b)
        pltpu.sync_copy(vals_hbm.at[s, pl.ds(g * G, G), :], vals_b)

        def chunk_body(k, _):
            pltpu.async_copy(x_hbm.at[cols_b.at[k]], xrows_v, sem).wait()

            def edge_body(e, _):
                val16 = plsc.load_gather(
                    vals_b, [jnp.full((16,), k, jnp.int32),
                             jnp.full((16,), e, jnp.int32)])
                for f in range(nvec):
                    xv = xrows_v[e, pl.ds(f * 16, 16)]
                    xs = jnp.where(sq, xv * xv, xv)
                    scaled_v[e, pl.ds(f * 16, 16)] = xs * val16
                return 0
            lax.fori_loop(0, K, edge_body, 0)

            pltpu.sync_copy(scaled_v, agg_sh.at[rows_b.at[k]], add=True)
            return 0
        lax.fori_loop(0, G, chunk_body, 0)
        return 0
    lax.fori_loop(0, ngroups, group_body, 0)

    plsc.subcore_barrier()
    pltpu.sync_copy(agg_sh.at[pl.ds(s * rows_per_sub, rows_per_sub), :],
                    out_hbm.at[c, pl.ds(s * rows_per_sub, rows_per_sub), :])
    @pl.when(s == NS - 1)
    def _():
        pltpu.sync_copy(agg_sh.at[pl.ds(rows_per_sub * NS, tail), :],
                        out_hbm.at[c, pl.ds(rows_per_sub * NS, tail), :])


def _sc_agg(x, rows3, cols3, vals3):
    n, d = x.shape
    nchunks = rows3.shape[1]
    mesh = plsc.VectorSubcoreMesh(core_axis_name="c", subcore_axis_name="s")
    fn = pl.kernel(
        _sc_agg_body,
        out_type=jax.ShapeDtypeStruct((NC, n, d), jnp.float32),
        mesh=mesh,
        compiler_params=pltpu.CompilerParams(needs_layout_passes=False),
        scratch_types=[
            pltpu.VMEM_SHARED((n, d), jnp.float32),
            pltpu.VMEM((G, K), jnp.int32),
            pltpu.VMEM((G, K), jnp.int32),
            pltpu.VMEM((G, K), jnp.float32),
            pltpu.VMEM((K, d), jnp.float32),
            pltpu.VMEM((K, d), jnp.float32),
            pltpu.SemaphoreType.DMA,
        ],
    )
    return fn(x, rows3, cols3, vals3)


def _selu(z):
    return _SELU_SCALE * jnp.where(z > 0, z, _SELU_ALPHA * (jnp.exp(z) - 1.0))


def _tc_mlp_body(x_ref, agg_ref, w00, b00, w01, b01, w10, b10, w11, b11, out_ref):
    xb = x_ref[...]
    params = [((w00, b00), (w01, b01)), ((w10, b10), (w11, b11))]
    d = xb.shape[1]
    cur = xb
    for i in range(2):
        t = agg_ref[i] + cur
        for (w, b) in params[i]:
            t = _selu(jnp.dot(t, w[...], precision=lax.Precision.HIGHEST,
                              preferred_element_type=jnp.float32) + b[...])
        out_ref[:, i * d:(i + 1) * d] = t
        cur = xb * xb


def _tc_mlp(x, agg, weights):
    n, d = x.shape
    bn = 1000
    grid = (n // bn,)
    full = lambda shape: pl.BlockSpec(shape, lambda i: (0,) * len(shape))
    in_specs = [
        pl.BlockSpec((bn, d), lambda i: (i, 0)),
        pl.BlockSpec((NC, bn, d), lambda i: (0, i, 0)),
    ]
    for (w, b) in weights:
        in_specs.append(full(w.shape))
        in_specs.append(full(b.shape))
    flat = []
    for (w, b) in weights:
        flat += [w, b]
    return pl.pallas_call(
        _tc_mlp_body,
        grid=grid,
        in_specs=in_specs,
        out_specs=pl.BlockSpec((bn, 2 * d), lambda i: (i, 0)),
        out_shape=jax.ShapeDtypeStruct((n, 2 * d), jnp.float32),
    )(x, agg, *flat)


def kernel(x, edge_index, edge_vals, W_0_0, b_0_0, W_0_1, b_0_1, W_1_0, b_1_0, W_1_1, b_1_1):
    n, d = x.shape
    e = edge_vals.shape[0]
    quantum = NS * G * K
    ep = (e + quantum - 1) // quantum * quantum
    pad = ep - e
    rows_f = edge_index[0]
    cols_f = edge_index[1]
    vals_f = edge_vals
    if pad:
        zi = jnp.zeros((pad,), jnp.int32)
        rows_f = jnp.concatenate([rows_f, zi])
        cols_f = jnp.concatenate([cols_f, zi])
        vals_f = jnp.concatenate([vals_f, jnp.zeros((pad,), jnp.float32)])
    nchunks = ep // (NS * K)
    rows3 = rows_f.reshape(NS, nchunks, K)
    cols3 = cols_f.reshape(NS, nchunks, K)
    vals3 = vals_f.reshape(NS, nchunks, K)
    agg = _sc_agg(x, rows3, cols3, vals3)
    weights = [
        (W_0_0, b_0_0.reshape(1, d)), (W_0_1, b_0_1.reshape(1, d)),
        (W_1_0, b_1_0.reshape(1, d)), (W_1_1, b_1_1.reshape(1, d)),
    ]
    return _tc_mlp(x, agg, weights)
